# baseline (speedup 1.0000x reference)
import functools

import jax
import jax.numpy as jnp
from jax import lax
from jax.experimental import pallas as pl
from jax.experimental.pallas import tpu as pltpu
from jax.experimental.pallas import tpu_sc as plsc

_TC_ROWS = 40960


def _copy_block_kernel(x_ref, o_ref):
    o_ref[...] = x_ref[...]


def _tc_copy(x2):
    rows, F = x2.shape
    block_rows = 8192
    return pl.pallas_call(
        _copy_block_kernel,
        grid=(rows // block_rows,),
        in_specs=[pl.BlockSpec((block_rows, F), lambda i: (i, 0))],
        out_specs=pl.BlockSpec((block_rows, F), lambda i: (i, 0)),
        out_shape=jax.ShapeDtypeStruct((rows, F), x2.dtype),
    )(x2)


def _sc_copy(x2):
    rows, F = x2.shape
    info = plsc.get_sparse_core_info()
    NC, NS = info.num_cores, info.num_subcores
    NW = NC * NS
    rpw = rows // NW
    chunk = 128
    n_chunks = rpw // chunk

    mesh = plsc.VectorSubcoreMesh(core_axis_name="c", subcore_axis_name="s")

    @functools.partial(
        pl.kernel,
        mesh=mesh,
        out_type=jax.ShapeDtypeStruct((rows, F), x2.dtype),
        scratch_types=[
            pltpu.VMEM((chunk, F), jnp.float32),
            pltpu.VMEM((chunk, F), jnp.float32),
            pltpu.SemaphoreType.DMA,
            pltpu.SemaphoreType.DMA,
        ],
    )
    def sc_copy(x_hbm, out_hbm, buf0, buf1, sem0, sem1):
        wid = lax.axis_index("s") * NC + lax.axis_index("c")
        base = wid * rpw
        bufs = (buf0, buf1)
        sems = (sem0, sem1)

        def out_copy(i):
            off = base + i * chunk
            return pltpu.make_async_copy(
                bufs[i % 2], out_hbm.at[pl.ds(off, chunk), :], sems[i % 2]
            )

        for i in range(n_chunks):
            if i >= 2:
                out_copy(i - 2).wait()
            off = base + i * chunk
            pltpu.sync_copy(x_hbm.at[pl.ds(off, chunk), :], bufs[i % 2])
            out_copy(i).start()
        out_copy(n_chunks - 2).wait()
        out_copy(n_chunks - 1).wait()

    return sc_copy(x2)


def kernel(x):
    B, T, F = x.shape
    rows = B * T
    x2 = x.reshape(rows, F)
    top = _tc_copy(x2[:_TC_ROWS])
    bot = _sc_copy(x2[_TC_ROWS:])
    return jnp.concatenate([top, bot], axis=0).reshape(B, T, F)


# speedup vs baseline: 2.3630x; 2.3630x over previous
import jax
import jax.numpy as jnp
from jax.experimental import pallas as pl
from jax.experimental.pallas import tpu as pltpu

_CHUNK = 4096
_NBUF = 3


def _dma_ring_kernel(x_hbm, o_hbm, *scratch):
    bufs = scratch[:_NBUF]
    in_sems = scratch[_NBUF : 2 * _NBUF]
    out_sems = scratch[2 * _NBUF :]
    rows = x_hbm.shape[0]
    n_chunks = rows // _CHUNK

    def in_copy(i):
        b = i % _NBUF
        return pltpu.make_async_copy(
            x_hbm.at[pl.ds(i * _CHUNK, _CHUNK), :], bufs[b], in_sems[b]
        )

    def out_copy(i):
        b = i % _NBUF
        return pltpu.make_async_copy(
            bufs[b], o_hbm.at[pl.ds(i * _CHUNK, _CHUNK), :], out_sems[b]
        )

    in_copy(0).start()
    for i in range(n_chunks):
        in_copy(i).wait()
        out_copy(i).start()
        if i + 1 < n_chunks:
            if i + 1 >= _NBUF:
                out_copy(i + 1 - _NBUF).wait()
            in_copy(i + 1).start()
    for i in range(max(0, n_chunks - _NBUF), n_chunks):
        out_copy(i).wait()


def kernel(x):
    B, T, F = x.shape
    rows = B * T
    x2 = x.reshape(rows, F)
    out = pl.pallas_call(
        _dma_ring_kernel,
        in_specs=[pl.BlockSpec(memory_space=pl.ANY)],
        out_specs=pl.BlockSpec(memory_space=pl.ANY),
        out_shape=jax.ShapeDtypeStruct((rows, F), x.dtype),
        scratch_shapes=(
            [pltpu.VMEM((_CHUNK, F), jnp.float32)] * _NBUF
            + [pltpu.SemaphoreType.DMA] * (2 * _NBUF)
        ),
    )(x2)
    return out.reshape(B, T, F)


# speedup vs baseline: 3.1619x; 1.3381x over previous
import jax
import jax.numpy as jnp
from jax.experimental import pallas as pl
from jax.experimental.pallas import tpu as pltpu

_CHUNK = 2048
_NBUF = 4


def _dma_ring_kernel(x_hbm, o_hbm, *scratch):
    bufs = scratch[:_NBUF]
    in_sems = scratch[_NBUF : 2 * _NBUF]
    out_sems = scratch[2 * _NBUF :]
    rows = x_hbm.shape[0]
    n_chunks = rows // _CHUNK

    def in_copy(i):
        b = i % _NBUF
        return pltpu.make_async_copy(
            x_hbm.at[pl.ds(i * _CHUNK, _CHUNK), :], bufs[b], in_sems[b]
        )

    def out_copy(i):
        b = i % _NBUF
        return pltpu.make_async_copy(
            bufs[b], o_hbm.at[pl.ds(i * _CHUNK, _CHUNK), :], out_sems[b]
        )

    in_copy(0).start()
    in_copy(1).start()
    for i in range(n_chunks):
        in_copy(i).wait()
        out_copy(i).start()
        if i + 2 < n_chunks:
            if i - 2 >= 0:
                out_copy(i - 2).wait()
            in_copy(i + 2).start()
    for i in range(max(0, n_chunks - _NBUF), n_chunks):
        out_copy(i).wait()


def kernel(x):
    B, T, F = x.shape
    rows = B * T
    x2 = x.reshape(rows, F)
    out = pl.pallas_call(
        _dma_ring_kernel,
        in_specs=[pl.BlockSpec(memory_space=pl.ANY)],
        out_specs=pl.BlockSpec(memory_space=pl.ANY),
        out_shape=jax.ShapeDtypeStruct((rows, F), x.dtype),
        scratch_shapes=(
            [pltpu.VMEM((_CHUNK, F), jnp.float32)] * _NBUF
            + [pltpu.SemaphoreType.DMA] * (2 * _NBUF)
        ),
    )(x2)
    return out.reshape(B, T, F)


# speedup vs baseline: 3.4712x; 1.0978x over previous
import jax
from jax.experimental import pallas as pl


def _copy_kernel(x_ref, o_ref):
    o_ref[...] = x_ref[...]


def kernel(x):
    B, T, F = x.shape
    rows = B * T
    x2 = x.reshape(rows, F)
    block_rows = 8192
    out = pl.pallas_call(
        _copy_kernel,
        grid=(rows // block_rows,),
        in_specs=[pl.BlockSpec((block_rows, F), lambda i: (i, 0))],
        out_specs=pl.BlockSpec((block_rows, F), lambda i: (i, 0)),
        out_shape=jax.ShapeDtypeStruct((rows, F), x.dtype),
    )(x2)
    return out.reshape(B, T, F)
